# baseline (speedup 1.0000x reference)
import functools

import jax
import jax.numpy as jnp
from jax import lax
from jax.experimental import pallas as pl
from jax.experimental.pallas import tpu as pltpu
from jax.experimental.pallas import tpu_sc as plsc

N_NODES = 16384
N_EDGES = 262144
NSUB = 16
NCORE = 2
EPT = N_EDGES // NSUB
SBLK = 4096


def _make_sc_agg(D, npass, chunk):
  G = D // 128
  R = N_NODES // (NCORE * npass)
  RPT = R // NSUB
  cap = SBLK + 3 * chunk
  mesh = plsc.VectorSubcoreMesh(core_axis_name="c", subcore_axis_name="s")

  @functools.partial(
      pl.kernel,
      out_type=jax.ShapeDtypeStruct((N_NODES, G, 128), jnp.float32),
      mesh=mesh,
      compiler_params=pltpu.CompilerParams(needs_layout_passes=False),
      scratch_types=[
          pltpu.VMEM((SBLK,), jnp.int32),
          pltpu.VMEM((SBLK,), jnp.int32),
          pltpu.VMEM((cap,), jnp.int32),
          pltpu.VMEM((cap,), jnp.int32),
          pltpu.VMEM((2, chunk), jnp.int32),
          pltpu.VMEM((2, chunk), jnp.int32),
          pltpu.VMEM((2, chunk, G, 128), jnp.float32),
          pltpu.VMEM_SHARED((R + 8, G, 128), jnp.float32),
          pltpu.SemaphoreType.DMA,
          pltpu.SemaphoreType.DMA,
      ],
  )
  def agg(x_hbm, src_hbm, dst_hbm, out_hbm,
          src_st, dst_st, src_cp, dst_cp, src_fx, dst_fx, rows, acc,
          sem0, sem1):
    cid = lax.axis_index("c")
    sid = lax.axis_index("s")
    zeros = jnp.zeros((16,), jnp.int32)
    dummy = jnp.full((16,), R, jnp.int32)
    sems = (sem0, sem1)

    def fire(j, slot):
      for t in range(chunk // 16):
        src_fx[slot, pl.ds(t * 16, 16)] = src_cp[pl.ds(j * chunk + t * 16, 16)]
        dst_fx[slot, pl.ds(t * 16, 16)] = dst_cp[pl.ds(j * chunk + t * 16, 16)]
      pltpu.async_copy(x_hbm.at[src_fx.at[slot]], rows.at[slot], sems[slot])

    def drain_add(slot):
      pltpu.make_async_copy(x_hbm.at[src_fx.at[slot]], rows.at[slot],
                            sems[slot]).wait()
      pltpu.sync_copy(rows.at[slot], acc.at[dst_fx.at[slot]], add=True)

    for p in range(npass):
      lo = (p * NCORE + cid) * R
      pltpu.sync_copy(x_hbm.at[pl.ds(lo + sid * RPT, RPT)],
                      acc.at[pl.ds(sid * RPT, RPT)])
      plsc.subcore_barrier()

      for b in range(EPT // SBLK):
        ebase = sid * EPT + b * SBLK
        pltpu.sync_copy(src_hbm.at[pl.ds(ebase, SBLK)], src_st)
        pltpu.sync_copy(dst_hbm.at[pl.ds(ebase, SBLK)], dst_st)

        def cbody(i, off):
          d = dst_st[pl.ds(i * 16, 16)]
          s = src_st[pl.ds(i * 16, 16)]
          m = (d >= lo) & (d < lo + R)
          mi = m.astype(jnp.int32)
          pos = off + plsc.cumsum(mi) - 1
          plsc.store_scatter(dst_cp, [pos], d - lo, mask=m)
          plsc.store_scatter(src_cp, [pos], s, mask=m)
          return off + jnp.sum(mi)

        off = lax.fori_loop(0, SBLK // 16, cbody, jnp.int32(0))

        for t in range(2 * chunk // 16 + 1):
          dst_cp[pl.ds(off + t * 16, 16)] = dummy
          src_cp[pl.ds(off + t * 16, 16)] = zeros

        nch = 2 * ((off + (2 * chunk - 1)) // (2 * chunk))

        @pl.when(nch > 0)
        def _prime():
          fire(0, 0)

        def pair_body(q, c):
          j1 = 2 * q + 1
          fire(j1, 1)
          drain_add(0)

          @pl.when(j1 + 1 < nch)
          def _next():
            fire(j1 + 1, 0)

          drain_add(1)
          return c

        lax.fori_loop(0, nch // 2, pair_body, 0)

      plsc.subcore_barrier()
      pltpu.sync_copy(acc.at[pl.ds(sid * RPT, RPT)],
                      out_hbm.at[pl.ds(lo + sid * RPT, RPT)])

  return agg


_agg128 = _make_sc_agg(128, 1, 128)
_agg512 = _make_sc_agg(512, 8, 64)


def _mlp_body(x_ref, w1_ref, b1_ref, w2_ref, b2_ref, o_ref):
  h = jnp.dot(x_ref[...], w1_ref[...], preferred_element_type=jnp.float32)
  h = jnp.maximum(h + b1_ref[...], 0.0)
  o_ref[...] = (jnp.dot(h, w2_ref[...], preferred_element_type=jnp.float32)
                + b2_ref[...])


def _tc_mlp(x, w1, b1, w2, b2, bm=1024):
  n, d = x.shape
  dh = w1.shape[1]
  do = w2.shape[1]
  return pl.pallas_call(
      _mlp_body,
      grid=(n // bm,),
      in_specs=[
          pl.BlockSpec((bm, d), lambda i: (i, 0)),
          pl.BlockSpec((d, dh), lambda i: (0, 0)),
          pl.BlockSpec((1, dh), lambda i: (0, 0)),
          pl.BlockSpec((dh, do), lambda i: (0, 0)),
          pl.BlockSpec((1, do), lambda i: (0, 0)),
      ],
      out_specs=pl.BlockSpec((bm, do), lambda i: (i, 0)),
      out_shape=jax.ShapeDtypeStruct((n, do), jnp.float32),
  )(x, w1, b1.reshape(1, -1), w2, b2.reshape(1, -1))


def kernel(W, edge_index, BASIS, W1_0, b1_0, W2_0, b2_0, W1_1, b1_1, W2_1, b2_1):
  x0 = W.reshape(N_NODES, 1, 128)
  src = edge_index[0]
  dst = edge_index[1]
  eye = jnp.eye(16, dtype=jnp.float32)
  h0 = _agg128(x0, src, dst).reshape(N_NODES, 128)
  x1 = _tc_mlp(h0, jnp.kron(eye, W1_0), jnp.tile(b1_0, 16),
               jnp.kron(eye, W2_0), jnp.tile(b2_0, 16))
  h1 = _agg512(x1.reshape(N_NODES, 4, 128), src, dst).reshape(N_NODES, 512)
  pe = _tc_mlp(h1, jnp.kron(eye, W1_1), jnp.tile(b1_1, 16),
               jnp.tile(W2_1, (16, 1)), 16.0 * b2_1)
  return pe


# speedup vs baseline: 1.9502x; 1.9502x over previous
import functools

import jax
import jax.numpy as jnp
from jax import lax
from jax.experimental import pallas as pl
from jax.experimental.pallas import tpu as pltpu
from jax.experimental.pallas import tpu_sc as plsc

N_NODES = 16384
N_EDGES = 262144
NSUB = 16
NCORE = 2
EPT = N_EDGES // NSUB
SBLK = 4096


def _make_sc_agg(D, npass, chunk):
  G = D // 128
  R = N_NODES // (NCORE * npass)
  RPT = R // NSUB
  cap = SBLK + 2 * chunk
  mesh = plsc.VectorSubcoreMesh(core_axis_name="c", subcore_axis_name="s")

  @functools.partial(
      pl.kernel,
      out_type=jax.ShapeDtypeStruct((N_NODES, G, 128), jnp.float32),
      mesh=mesh,
      compiler_params=pltpu.CompilerParams(needs_layout_passes=False),
      scratch_types=[
          pltpu.VMEM((SBLK,), jnp.int32),
          pltpu.VMEM((SBLK,), jnp.int32),
          pltpu.VMEM((cap,), jnp.int32),
          pltpu.VMEM((cap,), jnp.int32),
          pltpu.VMEM((chunk,), jnp.int32),
          pltpu.VMEM((chunk,), jnp.int32),
          pltpu.VMEM((chunk, G, 128), jnp.float32),
          pltpu.VMEM_SHARED((R + 8, G, 128), jnp.float32),
          pltpu.SemaphoreType.DMA,
      ],
  )
  def agg(x_hbm, src_hbm, dst_hbm, out_hbm,
          src_st, dst_st, src_cp, dst_cp, src_fx, dst_fx, rows, acc, sem):
    cid = lax.axis_index("c")
    sid = lax.axis_index("s")
    zeros = jnp.zeros((16,), jnp.int32)
    dummy = jnp.full((16,), R, jnp.int32)

    for p in range(npass):
      lo = (p * NCORE + cid) * R
      pltpu.sync_copy(x_hbm.at[pl.ds(lo + sid * RPT, RPT)],
                      acc.at[pl.ds(sid * RPT, RPT)])
      plsc.subcore_barrier()

      for b in range(EPT // SBLK):
        ebase = sid * EPT + b * SBLK
        pltpu.sync_copy(src_hbm.at[pl.ds(ebase, SBLK)], src_st)
        pltpu.sync_copy(dst_hbm.at[pl.ds(ebase, SBLK)], dst_st)

        def cbody(i, off):
          d = dst_st[pl.ds(i * 16, 16)]
          s = src_st[pl.ds(i * 16, 16)]
          m = (d >= lo) & (d < lo + R)
          mi = m.astype(jnp.int32)
          pos = off + plsc.cumsum(mi) - 1
          plsc.store_scatter(dst_cp, [pos], d - lo, mask=m)
          plsc.store_scatter(src_cp, [pos], s, mask=m)
          return off + jnp.sum(mi)

        off = lax.fori_loop(0, SBLK // 16, cbody, jnp.int32(0))

        for t in range(chunk // 16 + 1):
          dst_cp[pl.ds(off + t * 16, 16)] = dummy
          src_cp[pl.ds(off + t * 16, 16)] = zeros

        nch = (off + (chunk - 1)) // chunk

        def gbody(j, c):
          for t in range(chunk // 16):
            src_fx[pl.ds(t * 16, 16)] = src_cp[pl.ds(j * chunk + t * 16, 16)]
            dst_fx[pl.ds(t * 16, 16)] = dst_cp[pl.ds(j * chunk + t * 16, 16)]
          pltpu.async_copy(x_hbm.at[src_fx], rows, sem).wait()
          pltpu.sync_copy(rows, acc.at[dst_fx], add=True)
          return c

        lax.fori_loop(0, nch, gbody, 0)

      plsc.subcore_barrier()
      pltpu.sync_copy(acc.at[pl.ds(sid * RPT, RPT)],
                      out_hbm.at[pl.ds(lo + sid * RPT, RPT)])

  return agg


_agg128 = _make_sc_agg(128, 1, 256)
_agg512 = _make_sc_agg(512, 4, 64)


def _mlp_body(x_ref, w1_ref, b1_ref, w2_ref, b2_ref, o_ref):
  h = jnp.dot(x_ref[...], w1_ref[...], preferred_element_type=jnp.float32)
  h = jnp.maximum(h + b1_ref[...], 0.0)
  o_ref[...] = (jnp.dot(h, w2_ref[...], preferred_element_type=jnp.float32)
                + b2_ref[...])


def _tc_mlp(x, w1, b1, w2, b2, bm=1024):
  n, d = x.shape
  dh = w1.shape[1]
  do = w2.shape[1]
  return pl.pallas_call(
      _mlp_body,
      grid=(n // bm,),
      in_specs=[
          pl.BlockSpec((bm, d), lambda i: (i, 0)),
          pl.BlockSpec((d, dh), lambda i: (0, 0)),
          pl.BlockSpec((1, dh), lambda i: (0, 0)),
          pl.BlockSpec((dh, do), lambda i: (0, 0)),
          pl.BlockSpec((1, do), lambda i: (0, 0)),
      ],
      out_specs=pl.BlockSpec((bm, do), lambda i: (i, 0)),
      out_shape=jax.ShapeDtypeStruct((n, do), jnp.float32),
  )(x, w1, b1.reshape(1, -1), w2, b2.reshape(1, -1))


def kernel(W, edge_index, BASIS, W1_0, b1_0, W2_0, b2_0, W1_1, b1_1, W2_1, b2_1):
  x0 = W.reshape(N_NODES, 1, 128)
  src = edge_index[0]
  dst = edge_index[1]
  eye = jnp.eye(16, dtype=jnp.float32)
  h0 = _agg128(x0, src, dst).reshape(N_NODES, 128)
  x1 = _tc_mlp(h0, jnp.kron(eye, W1_0), jnp.tile(b1_0, 16),
               jnp.kron(eye, W2_0), jnp.tile(b2_0, 16))
  h1 = _agg512(x1.reshape(N_NODES, 4, 128), src, dst).reshape(N_NODES, 512)
  pe = _tc_mlp(h1, jnp.kron(eye, W1_1), jnp.tile(b1_1, 16),
               jnp.tile(W2_1, (16, 1)), 16.0 * b2_1)
  return pe


# speedup vs baseline: 2.6624x; 1.3652x over previous
import functools

import jax
import jax.numpy as jnp
from jax import lax
from jax.experimental import pallas as pl
from jax.experimental.pallas import tpu as pltpu
from jax.experimental.pallas import tpu_sc as plsc

N_NODES = 16384
N_EDGES = 262144
NSUB = 16
NCORE = 2
EPT = N_EDGES // NSUB
SBLK = 4096


def _make_sc_agg(D, npass, chunk):
  G = D // 128
  R = N_NODES // (NCORE * npass)
  RPT = R // NSUB
  cap = SBLK + 2 * chunk
  mesh = plsc.VectorSubcoreMesh(core_axis_name="c", subcore_axis_name="s")

  @functools.partial(
      pl.kernel,
      out_type=jax.ShapeDtypeStruct((N_NODES, G, 128), jnp.float32),
      mesh=mesh,
      compiler_params=pltpu.CompilerParams(needs_layout_passes=False),
      scratch_types=[
          pltpu.VMEM((SBLK,), jnp.int32),
          pltpu.VMEM((SBLK,), jnp.int32),
          pltpu.VMEM((cap,), jnp.int32),
          pltpu.VMEM((cap,), jnp.int32),
          pltpu.VMEM((chunk,), jnp.int32),
          pltpu.VMEM((chunk,), jnp.int32),
          pltpu.VMEM((chunk, G, 128), jnp.float32),
          pltpu.VMEM_SHARED((R + 8, G, 128), jnp.float32),
          pltpu.SemaphoreType.DMA,
      ],
  )
  def agg(x_hbm, src_hbm, dst_hbm, out_hbm,
          src_st, dst_st, src_cp, dst_cp, src_fx, dst_fx, rows, acc, sem):
    cid = lax.axis_index("c")
    sid = lax.axis_index("s")
    zeros = jnp.zeros((16,), jnp.int32)
    dummy = jnp.full((16,), R, jnp.int32)

    for p in range(npass):
      lo = (p * NCORE + cid) * R
      pltpu.sync_copy(x_hbm.at[pl.ds(lo + sid * RPT, RPT)],
                      acc.at[pl.ds(sid * RPT, RPT)])
      plsc.subcore_barrier()

      for b in range(EPT // SBLK):
        ebase = sid * EPT + b * SBLK
        pltpu.sync_copy(src_hbm.at[pl.ds(ebase, SBLK)], src_st)
        pltpu.sync_copy(dst_hbm.at[pl.ds(ebase, SBLK)], dst_st)

        def cbody(i, off):
          d = dst_st[pl.ds(i * 16, 16)]
          s = src_st[pl.ds(i * 16, 16)]
          m = (d >= lo) & (d < lo + R)
          mi = m.astype(jnp.int32)
          pos = off + plsc.cumsum(mi) - 1
          plsc.store_scatter(dst_cp, [pos], d - lo, mask=m)
          plsc.store_scatter(src_cp, [pos], s, mask=m)
          return off + jnp.sum(mi)

        off = lax.fori_loop(0, SBLK // 16, cbody, jnp.int32(0))

        for t in range(chunk // 16 + 1):
          dst_cp[pl.ds(off + t * 16, 16)] = dummy
          src_cp[pl.ds(off + t * 16, 16)] = zeros

        nch = (off + (chunk - 1)) // chunk

        def gbody(j, c):
          for t in range(chunk // 16):
            src_fx[pl.ds(t * 16, 16)] = src_cp[pl.ds(j * chunk + t * 16, 16)]
            dst_fx[pl.ds(t * 16, 16)] = dst_cp[pl.ds(j * chunk + t * 16, 16)]
          pltpu.async_copy(x_hbm.at[src_fx], rows, sem).wait()
          pltpu.sync_copy(rows, acc.at[dst_fx], add=True)
          return c

        lax.fori_loop(0, nch, gbody, 0)

      plsc.subcore_barrier()
      pltpu.sync_copy(acc.at[pl.ds(sid * RPT, RPT)],
                      out_hbm.at[pl.ds(lo + sid * RPT, RPT)])

  return agg


_agg128 = _make_sc_agg(128, 1, 128)
_agg512 = _make_sc_agg(512, 4, 32)


def _mlp_body(x_ref, w1_ref, b1_ref, w2_ref, b2_ref, o_ref):
  h = jnp.dot(x_ref[...], w1_ref[...], preferred_element_type=jnp.float32)
  h = jnp.maximum(h + b1_ref[...], 0.0)
  o_ref[...] = (jnp.dot(h, w2_ref[...], preferred_element_type=jnp.float32)
                + b2_ref[...])


def _tc_mlp(x, w1, b1, w2, b2, bm=1024):
  n, d = x.shape
  dh = w1.shape[1]
  do = w2.shape[1]
  return pl.pallas_call(
      _mlp_body,
      grid=(n // bm,),
      in_specs=[
          pl.BlockSpec((bm, d), lambda i: (i, 0)),
          pl.BlockSpec((d, dh), lambda i: (0, 0)),
          pl.BlockSpec((1, dh), lambda i: (0, 0)),
          pl.BlockSpec((dh, do), lambda i: (0, 0)),
          pl.BlockSpec((1, do), lambda i: (0, 0)),
      ],
      out_specs=pl.BlockSpec((bm, do), lambda i: (i, 0)),
      out_shape=jax.ShapeDtypeStruct((n, do), jnp.float32),
  )(x, w1, b1.reshape(1, -1), w2, b2.reshape(1, -1))


def kernel(W, edge_index, BASIS, W1_0, b1_0, W2_0, b2_0, W1_1, b1_1, W2_1, b2_1):
  x0 = W.reshape(N_NODES, 1, 128)
  src = edge_index[0]
  dst = edge_index[1]
  eye = jnp.eye(16, dtype=jnp.float32)
  h0 = _agg128(x0, src, dst).reshape(N_NODES, 128)
  x1 = _tc_mlp(h0, jnp.kron(eye, W1_0), jnp.tile(b1_0, 16),
               jnp.kron(eye, W2_0), jnp.tile(b2_0, 16))
  h1 = _agg512(x1.reshape(N_NODES, 4, 128), src, dst).reshape(N_NODES, 512)
  pe = _tc_mlp(h1, jnp.kron(eye, W1_1), jnp.tile(b1_1, 16),
               jnp.tile(W2_1, (16, 1)), 16.0 * b2_1)
  return pe


# speedup vs baseline: 2.7352x; 1.0273x over previous
import functools

import jax
import jax.numpy as jnp
from jax import lax
from jax.experimental import pallas as pl
from jax.experimental.pallas import tpu as pltpu
from jax.experimental.pallas import tpu_sc as plsc

N_NODES = 16384
N_EDGES = 262144
NSUB = 16
NCORE = 2
EPT = N_EDGES // NSUB
SBLK = 4096


def _make_sc_agg(D, npass, chunk):
  G = D // 128
  R = N_NODES // (NCORE * npass)
  RPT = R // NSUB
  cap = SBLK + 2 * chunk
  mesh = plsc.VectorSubcoreMesh(core_axis_name="c", subcore_axis_name="s")

  @functools.partial(
      pl.kernel,
      out_type=jax.ShapeDtypeStruct((N_NODES, G, 128), jnp.float32),
      mesh=mesh,
      compiler_params=pltpu.CompilerParams(needs_layout_passes=False),
      scratch_types=[
          pltpu.VMEM((SBLK,), jnp.int32),
          pltpu.VMEM((SBLK,), jnp.int32),
          pltpu.VMEM((cap,), jnp.int32),
          pltpu.VMEM((cap,), jnp.int32),
          pltpu.VMEM((chunk,), jnp.int32),
          pltpu.VMEM((chunk,), jnp.int32),
          pltpu.VMEM((chunk, G, 128), jnp.float32),
          pltpu.VMEM_SHARED((R + 8, G, 128), jnp.float32),
          pltpu.SemaphoreType.DMA,
      ],
  )
  def agg(x_hbm, src_hbm, dst_hbm, out_hbm,
          src_st, dst_st, src_cp, dst_cp, src_fx, dst_fx, rows, acc, sem):
    cid = lax.axis_index("c")
    sid = lax.axis_index("s")
    zeros = jnp.zeros((16,), jnp.int32)
    dummy = jnp.full((16,), R, jnp.int32)

    for p in range(npass):
      lo = (p * NCORE + cid) * R
      pltpu.sync_copy(x_hbm.at[pl.ds(lo + sid * RPT, RPT)],
                      acc.at[pl.ds(sid * RPT, RPT)])
      plsc.subcore_barrier()

      for b in range(EPT // SBLK):
        ebase = sid * EPT + b * SBLK
        pltpu.sync_copy(src_hbm.at[pl.ds(ebase, SBLK)], src_st)
        pltpu.sync_copy(dst_hbm.at[pl.ds(ebase, SBLK)], dst_st)

        def cbody(i, off):
          d = dst_st[pl.ds(i * 16, 16)]
          s = src_st[pl.ds(i * 16, 16)]
          m = (d >= lo) & (d < lo + R)
          mi = m.astype(jnp.int32)
          pos = off + plsc.cumsum(mi) - 1
          plsc.store_scatter(dst_cp, [pos], d - lo, mask=m)
          plsc.store_scatter(src_cp, [pos], s, mask=m)
          return off + jnp.sum(mi)

        off = lax.fori_loop(0, SBLK // 16, cbody, jnp.int32(0))

        for t in range(chunk // 16 + 1):
          dst_cp[pl.ds(off + t * 16, 16)] = dummy
          src_cp[pl.ds(off + t * 16, 16)] = zeros

        nch = (off + (chunk - 1)) // chunk

        def gbody(j, c):
          for t in range(chunk // 16):
            src_fx[pl.ds(t * 16, 16)] = src_cp[pl.ds(j * chunk + t * 16, 16)]
            dst_fx[pl.ds(t * 16, 16)] = dst_cp[pl.ds(j * chunk + t * 16, 16)]
          pltpu.async_copy(x_hbm.at[src_fx], rows, sem).wait()
          pltpu.sync_copy(rows, acc.at[dst_fx], add=True)
          return c

        lax.fori_loop(0, nch, gbody, 0)

      plsc.subcore_barrier()
      pltpu.sync_copy(acc.at[pl.ds(sid * RPT, RPT)],
                      out_hbm.at[pl.ds(lo + sid * RPT, RPT)])

  return agg


_agg128 = _make_sc_agg(128, 1, 64)
_agg512 = _make_sc_agg(512, 4, 16)


def _mlp_body(x_ref, w1_ref, b1_ref, w2_ref, b2_ref, o_ref):
  h = jnp.dot(x_ref[...], w1_ref[...], preferred_element_type=jnp.float32)
  h = jnp.maximum(h + b1_ref[...], 0.0)
  o_ref[...] = (jnp.dot(h, w2_ref[...], preferred_element_type=jnp.float32)
                + b2_ref[...])


def _tc_mlp(x, w1, b1, w2, b2, bm=1024):
  n, d = x.shape
  dh = w1.shape[1]
  do = w2.shape[1]
  return pl.pallas_call(
      _mlp_body,
      grid=(n // bm,),
      in_specs=[
          pl.BlockSpec((bm, d), lambda i: (i, 0)),
          pl.BlockSpec((d, dh), lambda i: (0, 0)),
          pl.BlockSpec((1, dh), lambda i: (0, 0)),
          pl.BlockSpec((dh, do), lambda i: (0, 0)),
          pl.BlockSpec((1, do), lambda i: (0, 0)),
      ],
      out_specs=pl.BlockSpec((bm, do), lambda i: (i, 0)),
      out_shape=jax.ShapeDtypeStruct((n, do), jnp.float32),
  )(x, w1, b1.reshape(1, -1), w2, b2.reshape(1, -1))


def kernel(W, edge_index, BASIS, W1_0, b1_0, W2_0, b2_0, W1_1, b1_1, W2_1, b2_1):
  x0 = W.reshape(N_NODES, 1, 128)
  src = edge_index[0]
  dst = edge_index[1]
  eye = jnp.eye(16, dtype=jnp.float32)
  h0 = _agg128(x0, src, dst).reshape(N_NODES, 128)
  x1 = _tc_mlp(h0, jnp.kron(eye, W1_0), jnp.tile(b1_0, 16),
               jnp.kron(eye, W2_0), jnp.tile(b2_0, 16))
  h1 = _agg512(x1.reshape(N_NODES, 4, 128), src, dst).reshape(N_NODES, 512)
  pe = _tc_mlp(h1, jnp.kron(eye, W1_1), jnp.tile(b1_1, 16),
               jnp.tile(W2_1, (16, 1)), 16.0 * b2_1)
  return pe


# speedup vs baseline: 2.7370x; 1.0006x over previous
import functools

import jax
import jax.numpy as jnp
from jax import lax
from jax.experimental import pallas as pl
from jax.experimental.pallas import tpu as pltpu
from jax.experimental.pallas import tpu_sc as plsc

N_NODES = 16384
N_EDGES = 262144
NSUB = 16
NCORE = 2
EPT = N_EDGES // NSUB
SBLK = 4096


def _make_sc_agg(D, npass, chunk):
  G = D // 128
  R = N_NODES // (NCORE * npass)
  RPT = R // NSUB
  cap = SBLK + 2 * chunk + 16
  crows = cap // chunk + 1
  shift = chunk.bit_length() - 1
  mesh = plsc.VectorSubcoreMesh(core_axis_name="c", subcore_axis_name="s")

  @functools.partial(
      pl.kernel,
      out_type=jax.ShapeDtypeStruct((N_NODES, G, 128), jnp.float32),
      mesh=mesh,
      compiler_params=pltpu.CompilerParams(needs_layout_passes=False),
      scratch_types=[
          pltpu.VMEM((SBLK,), jnp.int32),
          pltpu.VMEM((SBLK,), jnp.int32),
          pltpu.VMEM((cap,), jnp.int32),
          pltpu.VMEM((crows, chunk), jnp.int32),
          pltpu.VMEM((chunk, G, 128), jnp.float32),
          pltpu.VMEM_SHARED((R + 8, G, 128), jnp.float32),
          pltpu.SemaphoreType.DMA,
      ],
  )
  def agg(x_hbm, src_hbm, dst_hbm, out_hbm,
          src_st, dst_st, src_cp, dst_cp, rows, acc, sem):
    cid = lax.axis_index("c")
    sid = lax.axis_index("s")
    zeros = jnp.zeros((16,), jnp.int32)
    dummy = jnp.full((16,), R, jnp.int32)
    lane = lax.iota(jnp.int32, 16)

    for p in range(npass):
      lo = (p * NCORE + cid) * R
      pltpu.sync_copy(x_hbm.at[pl.ds(lo + sid * RPT, RPT)],
                      acc.at[pl.ds(sid * RPT, RPT)])
      plsc.subcore_barrier()

      for b in range(EPT // SBLK):
        ebase = sid * EPT + b * SBLK
        pltpu.sync_copy(src_hbm.at[pl.ds(ebase, SBLK)], src_st)
        pltpu.sync_copy(dst_hbm.at[pl.ds(ebase, SBLK)], dst_st)

        def cbody(i, offv):
          d = dst_st[pl.ds(i * 16, 16)]
          s = src_st[pl.ds(i * 16, 16)]
          m = (d >= lo) & (d < lo + R)
          mi = m.astype(jnp.int32)
          pos = offv + plsc.cumsum(mi) - 1
          plsc.store_scatter(src_cp, [pos], s, mask=m)
          plsc.store_scatter(dst_cp,
                             [lax.shift_right_logical(pos, shift),
                              pos & (chunk - 1)], d - lo, mask=m)
          return offv + plsc.all_reduce_population_count(m)

        offv = lax.fori_loop(0, SBLK // 16, cbody, zeros)
        off = jnp.max(offv)

        for t in range(chunk // 16 + 1):
          pos = off + t * 16 + lane
          plsc.store_scatter(src_cp, [pos], zeros)
          plsc.store_scatter(dst_cp,
                             [lax.shift_right_logical(pos, shift),
                              pos & (chunk - 1)], dummy)

        nch = (off + (chunk - 1)) // chunk

        def gbody(j, c):
          pltpu.async_copy(
              x_hbm.at[src_cp.at[pl.ds(j * chunk, chunk)]], rows, sem).wait()
          pltpu.sync_copy(rows, acc.at[dst_cp.at[j]], add=True)
          return c

        lax.fori_loop(0, nch, gbody, 0)

      plsc.subcore_barrier()
      pltpu.sync_copy(acc.at[pl.ds(sid * RPT, RPT)],
                      out_hbm.at[pl.ds(lo + sid * RPT, RPT)])

  return agg


_agg128 = _make_sc_agg(128, 1, 64)
_agg512 = _make_sc_agg(512, 4, 16)


def _mlp_body(x_ref, w1_ref, b1_ref, w2_ref, b2_ref, o_ref):
  h = jnp.dot(x_ref[...], w1_ref[...], preferred_element_type=jnp.float32)
  h = jnp.maximum(h + b1_ref[...], 0.0)
  o_ref[...] = (jnp.dot(h, w2_ref[...], preferred_element_type=jnp.float32)
                + b2_ref[...])


def _tc_mlp(x, w1, b1, w2, b2, bm=1024):
  n, d = x.shape
  dh = w1.shape[1]
  do = w2.shape[1]
  return pl.pallas_call(
      _mlp_body,
      grid=(n // bm,),
      in_specs=[
          pl.BlockSpec((bm, d), lambda i: (i, 0)),
          pl.BlockSpec((d, dh), lambda i: (0, 0)),
          pl.BlockSpec((1, dh), lambda i: (0, 0)),
          pl.BlockSpec((dh, do), lambda i: (0, 0)),
          pl.BlockSpec((1, do), lambda i: (0, 0)),
      ],
      out_specs=pl.BlockSpec((bm, do), lambda i: (i, 0)),
      out_shape=jax.ShapeDtypeStruct((n, do), jnp.float32),
  )(x, w1, b1.reshape(1, -1), w2, b2.reshape(1, -1))


def kernel(W, edge_index, BASIS, W1_0, b1_0, W2_0, b2_0, W1_1, b1_1, W2_1, b2_1):
  x0 = W.reshape(N_NODES, 1, 128)
  src = edge_index[0]
  dst = edge_index[1]
  eye = jnp.eye(16, dtype=jnp.float32)
  h0 = _agg128(x0, src, dst).reshape(N_NODES, 128)
  x1 = _tc_mlp(h0, jnp.kron(eye, W1_0), jnp.tile(b1_0, 16),
               jnp.kron(eye, W2_0), jnp.tile(b2_0, 16))
  h1 = _agg512(x1.reshape(N_NODES, 4, 128), src, dst).reshape(N_NODES, 512)
  pe = _tc_mlp(h1, jnp.kron(eye, W1_1), jnp.tile(b1_1, 16),
               jnp.tile(W2_1, (16, 1)), 16.0 * b2_1)
  return pe
